# baseline (device time: 24197 ns/iter reference)
import jax
import jax.numpy as jnp
from jax import lax
from jax.experimental import pallas as pl
from jax.experimental.pallas import tpu as pltpu

N_DEV = 8
_RELAY = (
    (1, 6, 0, 768),
    (2, 5, 768, 768),
    (4, 3, 1536, 512),
)


def _peer(my, o):
    b = my ^ ((my >> 1) & 1)
    q = b ^ o
    return q ^ ((q >> 1) & 1)


def kernel(x, dy):
    k, m = x.shape
    _, n = dy.shape
    m_out = m // N_DEV

    def body(x_ref, dy_ref, out_ref, pc_ref, recv_ref, relay_ref,
             send_sems, recv_sems, rsend_sems, rrecv_sems):
        my = lax.axis_index("i")
        far = _peer(my, 7)

        barrier = pltpu.get_barrier_semaphore()
        for o in range(1, N_DEV):
            pl.semaphore_signal(
                barrier, inc=1, device_id=(_peer(my, o),),
                device_id_type=pl.DeviceIdType.MESH,
            )

        xb = x_ref[...].astype(jnp.bfloat16)
        db = dy_ref[...].astype(jnp.bfloat16)
        relay_sends = []
        sends = []
        m_half = m // 2
        c_half = N_DEV // 2
        for h in range(2):
            ph = lax.dot_general(
                xb[:, h * m_half:(h + 1) * m_half], db,
                (((0,), (0,)), ((), ())),
                preferred_element_type=jnp.float32,
            )
            pc_ref[h * c_half:(h + 1) * c_half] = (
                ph.astype(jnp.bfloat16).reshape(c_half, m_out, n)
            )
            if h == 0:
                pl.semaphore_wait(barrier, N_DEV - 1)
            in_half = lambda c: jnp.logical_and(
                h * c_half <= c, c < (h + 1) * c_half
            )

            for di, (o, _, g0, gw) in enumerate(_RELAY):
                rdma = pltpu.make_async_remote_copy(
                    src_ref=pc_ref.at[far, :, pl.ds(g0, gw)],
                    dst_ref=relay_ref.at[di, :, pl.ds(0, gw)],
                    send_sem=rsend_sems.at[di],
                    recv_sem=rrecv_sems.at[di],
                    device_id=(_peer(my, o),),
                    device_id_type=pl.DeviceIdType.MESH,
                )
                if h == 0:
                    relay_sends.append(rdma)

                @pl.when(in_half(far))
                def _():
                    rdma.start()

            for o in (1, 2, 4):
                dst = _peer(my, o)
                rdma = pltpu.make_async_remote_copy(
                    src_ref=pc_ref.at[dst],
                    dst_ref=recv_ref.at[my],
                    send_sem=send_sems.at[dst],
                    recv_sem=recv_sems.at[my],
                    device_id=(dst,),
                    device_id_type=pl.DeviceIdType.MESH,
                )
                if h == 0:
                    sends.append(rdma)

                @pl.when(in_half(dst))
                def _():
                    rdma.start()

        recv_ref[pl.ds(far, 1)] = jnp.zeros((1, m_out, n), jnp.bfloat16)
        recv_ref[pl.ds(my, 1)] = pc_ref[pl.ds(my, 1)]

        for di, (o, od, g0, gw) in enumerate(_RELAY):
            rrecv = pltpu.make_async_remote_copy(
                src_ref=relay_ref.at[di, :, pl.ds(0, gw)],
                dst_ref=relay_ref.at[di, :, pl.ds(0, gw)],
                send_sem=rsend_sems.at[di],
                recv_sem=rrecv_sems.at[di],
                device_id=(_peer(my, o),),
                device_id_type=pl.DeviceIdType.MESH,
            )
            rrecv.wait_recv()
            dst = _peer(my, od)
            pc_ref[pl.ds(dst, 1), :, pl.ds(g0, gw)] += (
                relay_ref[pl.ds(di, 1), :, pl.ds(0, gw)]
            )
            rdma = pltpu.make_async_remote_copy(
                src_ref=pc_ref.at[dst],
                dst_ref=recv_ref.at[my],
                send_sem=send_sems.at[dst],
                recv_sem=recv_sems.at[my],
                device_id=(dst,),
                device_id_type=pl.DeviceIdType.MESH,
            )
            rdma.start()
            sends.append(rdma)

        for o in (1, 2, 4, 3, 5, 6):
            s = _peer(my, o)
            recv = pltpu.make_async_remote_copy(
                src_ref=recv_ref.at[s],
                dst_ref=recv_ref.at[s],
                send_sem=send_sems.at[s],
                recv_sem=recv_sems.at[s],
                device_id=(s,),
                device_id_type=pl.DeviceIdType.MESH,
            )
            recv.wait_recv()

        out_ref[...] = jnp.sum(recv_ref[...].astype(jnp.float32), axis=0)

        for rdma in relay_sends + sends:
            rdma.wait_send()

    return pl.pallas_call(
        body,
        out_shape=jax.ShapeDtypeStruct((m_out, n), jnp.float32),
        in_specs=[
            pl.BlockSpec(memory_space=pltpu.VMEM),
            pl.BlockSpec(memory_space=pltpu.VMEM),
        ],
        out_specs=pl.BlockSpec(memory_space=pltpu.VMEM),
        scratch_shapes=[
            pltpu.VMEM((N_DEV, m_out, n), jnp.bfloat16),
            pltpu.VMEM((N_DEV, m_out, n), jnp.bfloat16),
            pltpu.VMEM((3, m_out, 768), jnp.bfloat16),
            pltpu.SemaphoreType.DMA((N_DEV,)),
            pltpu.SemaphoreType.DMA((N_DEV,)),
            pltpu.SemaphoreType.DMA((3,)),
            pltpu.SemaphoreType.DMA((3,)),
        ],
        compiler_params=pltpu.CompilerParams(collective_id=0),
    )(x, dy)


# device time: 21603 ns/iter; 1.1201x vs baseline; 1.1201x over previous
import jax
import jax.numpy as jnp
from jax import lax
from jax.experimental import pallas as pl
from jax.experimental.pallas import tpu as pltpu

N_DEV = 8
_RELAY = (
    (1, 6, 0, 768),
    (2, 5, 768, 768),
    (4, 3, 1536, 512),
)


def _peer(my, o):
    b = my ^ ((my >> 1) & 1)
    q = b ^ o
    return q ^ ((q >> 1) & 1)


def kernel(x, dy):
    k, m = x.shape
    _, n = dy.shape
    m_out = m // N_DEV

    def body(x_ref, dy_ref, out_ref, pc_ref, recv_ref, relay_ref,
             send_sems, recv_sems, rsend_sems, rrecv_sems):
        my = lax.axis_index("i")
        far = _peer(my, 7)

        barrier = pltpu.get_barrier_semaphore()
        for o in range(1, N_DEV):
            pl.semaphore_signal(
                barrier, inc=1, device_id=(_peer(my, o),),
                device_id_type=pl.DeviceIdType.MESH,
            )

        xb = x_ref[...].astype(jnp.bfloat16)
        db = dy_ref[...].astype(jnp.bfloat16)
        p = lax.dot_general(
            xb, db, (((0,), (0,)), ((), ())),
            preferred_element_type=jnp.float32,
        )
        pc_ref[...] = p.astype(jnp.bfloat16).reshape(N_DEV, m_out, n)

        pl.semaphore_wait(barrier, N_DEV - 1)

        relay_sends = []
        for di, (o, _, g0, gw) in enumerate(_RELAY):
            rdma = pltpu.make_async_remote_copy(
                src_ref=pc_ref.at[far, :, pl.ds(g0, gw)],
                dst_ref=relay_ref.at[di, :, pl.ds(0, gw)],
                send_sem=rsend_sems.at[di],
                recv_sem=rrecv_sems.at[di],
                device_id=(_peer(my, o),),
                device_id_type=pl.DeviceIdType.MESH,
            )
            rdma.start()
            relay_sends.append(rdma)

        sends = []
        for o in (1, 2, 4):
            dst = _peer(my, o)
            rdma = pltpu.make_async_remote_copy(
                src_ref=pc_ref.at[dst],
                dst_ref=recv_ref.at[my],
                send_sem=send_sems.at[dst],
                recv_sem=recv_sems.at[my],
                device_id=(dst,),
                device_id_type=pl.DeviceIdType.MESH,
            )
            rdma.start()
            sends.append(rdma)

        recv_ref[pl.ds(far, 1)] = jnp.zeros((1, m_out, n), jnp.bfloat16)
        recv_ref[pl.ds(my, 1)] = pc_ref[pl.ds(my, 1)]

        for di, (o, od, g0, gw) in enumerate(_RELAY):
            rrecv = pltpu.make_async_remote_copy(
                src_ref=relay_ref.at[di, :, pl.ds(0, gw)],
                dst_ref=relay_ref.at[di, :, pl.ds(0, gw)],
                send_sem=rsend_sems.at[di],
                recv_sem=rrecv_sems.at[di],
                device_id=(_peer(my, o),),
                device_id_type=pl.DeviceIdType.MESH,
            )
            rrecv.wait_recv()
            dst = _peer(my, od)
            pc_ref[pl.ds(dst, 1), :, pl.ds(g0, gw)] += (
                relay_ref[pl.ds(di, 1), :, pl.ds(0, gw)]
            )
            rdma = pltpu.make_async_remote_copy(
                src_ref=pc_ref.at[dst],
                dst_ref=recv_ref.at[my],
                send_sem=send_sems.at[dst],
                recv_sem=recv_sems.at[my],
                device_id=(dst,),
                device_id_type=pl.DeviceIdType.MESH,
            )
            rdma.start()
            sends.append(rdma)

        for o in (1, 2, 4, 3, 5, 6):
            s = _peer(my, o)
            recv = pltpu.make_async_remote_copy(
                src_ref=recv_ref.at[s],
                dst_ref=recv_ref.at[s],
                send_sem=send_sems.at[s],
                recv_sem=recv_sems.at[s],
                device_id=(s,),
                device_id_type=pl.DeviceIdType.MESH,
            )
            recv.wait_recv()

        out_ref[...] = jnp.sum(recv_ref[...].astype(jnp.float32), axis=0)

        for rdma in relay_sends + sends:
            rdma.wait_send()

    return pl.pallas_call(
        body,
        out_shape=jax.ShapeDtypeStruct((m_out, n), jnp.float32),
        in_specs=[
            pl.BlockSpec(memory_space=pltpu.VMEM),
            pl.BlockSpec(memory_space=pltpu.VMEM),
        ],
        out_specs=pl.BlockSpec(memory_space=pltpu.VMEM),
        scratch_shapes=[
            pltpu.VMEM((N_DEV, m_out, n), jnp.bfloat16),
            pltpu.VMEM((N_DEV, m_out, n), jnp.bfloat16),
            pltpu.VMEM((3, m_out, 768), jnp.bfloat16),
            pltpu.SemaphoreType.DMA((N_DEV,)),
            pltpu.SemaphoreType.DMA((N_DEV,)),
            pltpu.SemaphoreType.DMA((3,)),
            pltpu.SemaphoreType.DMA((3,)),
        ],
        compiler_params=pltpu.CompilerParams(collective_id=0),
    )(x, dy)
